# Initial kernel scaffold; baseline (speedup 1.0000x reference)
import jax
import jax.numpy as jnp
from jax import lax
from jax.experimental import pallas as pl
from jax.experimental.pallas import tpu as pltpu
from jax.experimental.pallas import tpu_sc as plsc

N = 2_000_000
R = 128
ROWS = N // 128
KC = 25
CHUNKS = ROWS // KC
NW = 32
ITERS = (CHUNKS + NW - 1) // NW


def _body(x_hbm, y_hbm, z_hbm, b_hbm, tab_hbm, out_hbm,
          xv, yv, zv, bv, idxv, gv, sem):
    w = lax.axis_index("s") * 2 + lax.axis_index("c")

    def chunk_iter(it, carry):
        c = w + NW * it

        @pl.when(c < CHUNKS)
        def _():
            rows = pl.ds(c * KC, KC)
            cps = [pltpu.async_copy(src.at[rows], dst, sem)
                   for src, dst in ((x_hbm, xv), (y_hbm, yv),
                                    (z_hbm, zv), (b_hbm, bv))]
            for cp in cps:
                cp.wait()

            def compute(i, carry2):
                r = i >> 3
                s = pl.ds((i & 7) * 16, 16)
                xf = xv[r, s]
                yf = yv[r, s]
                zf = zv[r, s]
                bi = bv[r, s]
                gx = ((xf * 0.5 + 0.5) * 128.0).astype(jnp.int32)
                gy = ((yf * 0.5 + 0.5) * 128.0).astype(jnp.int32)
                gz = ((zf * 0.5 + 0.5) * 128.0).astype(jnp.int32)
                gx = jnp.minimum(jnp.maximum(gx, 0), 127)
                gy = jnp.minimum(jnp.maximum(gy, 0), 127)
                gz = jnp.minimum(jnp.maximum(gz, 0), 127)
                f = ((bi * 128 + gx) * 128 + gy) * 128 + gz
                idxv[r, s] = f
                return carry2

            lax.fori_loop(0, KC * 8, compute, 0)

            gcps = [pltpu.async_copy(tab_hbm.at[idxv.at[r]], gv.at[r], sem)
                    for r in range(KC)]
            for cp in gcps:
                cp.wait()

            pltpu.sync_copy(gv, out_hbm.at[rows])

        return carry

    lax.fori_loop(0, ITERS, chunk_iter, 0)


def kernel(pts, bidx, occ_grid_per_batch, resolution):
    x = pts[:, 0].reshape(ROWS, 128)
    y = pts[:, 1].reshape(ROWS, 128)
    z = pts[:, 2].reshape(ROWS, 128)
    b2 = bidx.reshape(ROWS, 128)
    tab = occ_grid_per_batch.reshape(-1)

    mesh = plsc.VectorSubcoreMesh(core_axis_name="c", subcore_axis_name="s")
    out = pl.kernel(
        _body,
        out_type=jax.ShapeDtypeStruct((ROWS, 128), jnp.bool_),
        mesh=mesh,
        scratch_types=[
            pltpu.VMEM((KC, 128), jnp.float32),
            pltpu.VMEM((KC, 128), jnp.float32),
            pltpu.VMEM((KC, 128), jnp.float32),
            pltpu.VMEM((KC, 128), jnp.int32),
            pltpu.VMEM((KC, 128), jnp.int32),
            pltpu.VMEM((KC, 128), jnp.bool_),
            pltpu.SemaphoreType.DMA,
        ],
    )(x, y, z, b2, tab)
    return out.reshape(N)


# speedup vs baseline: 1.0540x; 1.0540x over previous
import jax
import jax.numpy as jnp
from jax import lax
from jax.experimental import pallas as pl
from jax.experimental.pallas import tpu as pltpu
from jax.experimental.pallas import tpu_sc as plsc

N = 2_000_000
R = 128
KC = 25
C = KC * 128
CHUNKS = N // C
NW = 32
ITERS = (CHUNKS + NW - 1) // NW


def _body(x_hbm, y_hbm, z_hbm, b_hbm, tab_hbm, out_hbm,
          xv, yv, zv, bv, idxv, gv, sem):
    w = lax.axis_index("s") * 2 + lax.axis_index("c")

    def chunk_iter(it, carry):
        c = w + NW * it

        @pl.when(c < CHUNKS)
        def _():
            sl = pl.ds(c * C, C)
            cps = [pltpu.async_copy(src.at[sl], dst, sem)
                   for src, dst in ((x_hbm, xv), (y_hbm, yv),
                                    (z_hbm, zv), (b_hbm, bv))]
            for cp in cps:
                cp.wait()

            def compute(i, carry2):
                s = pl.ds(i * 16, 16)
                xf = xv[s]
                yf = yv[s]
                zf = zv[s]
                bi = bv[s]
                gx = ((xf * 0.5 + 0.5) * 128.0).astype(jnp.int32)
                gy = ((yf * 0.5 + 0.5) * 128.0).astype(jnp.int32)
                gz = ((zf * 0.5 + 0.5) * 128.0).astype(jnp.int32)
                gx = jnp.minimum(jnp.maximum(gx, 0), 127)
                gy = jnp.minimum(jnp.maximum(gy, 0), 127)
                gz = jnp.minimum(jnp.maximum(gz, 0), 127)
                f = ((bi * 128 + gx) * 128 + gy) * 128 + gz
                idxv[i >> 3, pl.ds((i & 7) * 16, 16)] = f
                return carry2

            lax.fori_loop(0, C // 16, compute, 0)

            gcps = [pltpu.async_copy(tab_hbm.at[idxv.at[r]],
                                     gv.at[pl.ds(r * 128, 128)], sem)
                    for r in range(KC)]
            for cp in gcps:
                cp.wait()

            pltpu.sync_copy(gv, out_hbm.at[sl])

        return carry

    lax.fori_loop(0, ITERS, chunk_iter, 0)


def kernel(pts, bidx, occ_grid_per_batch, resolution):
    x = pts[:, 0]
    y = pts[:, 1]
    z = pts[:, 2]
    tab = occ_grid_per_batch.reshape(-1)

    mesh = plsc.VectorSubcoreMesh(core_axis_name="c", subcore_axis_name="s")
    out = pl.kernel(
        _body,
        out_type=jax.ShapeDtypeStruct((N,), jnp.bool_),
        mesh=mesh,
        scratch_types=[
            pltpu.VMEM((C,), jnp.float32),
            pltpu.VMEM((C,), jnp.float32),
            pltpu.VMEM((C,), jnp.float32),
            pltpu.VMEM((C,), jnp.int32),
            pltpu.VMEM((KC, 128), jnp.int32),
            pltpu.VMEM((C,), jnp.bool_),
            pltpu.SemaphoreType.DMA,
        ],
    )(x, y, z, bidx, tab)
    return out


# speedup vs baseline: 1.3549x; 1.2854x over previous
import jax
import jax.numpy as jnp
from jax import lax
from jax.experimental import pallas as pl
from jax.experimental.pallas import tpu as pltpu
from jax.experimental.pallas import tpu_sc as plsc

N = 2_000_000
R = 128
C = 3200
CHUNKS = N // C
NW = 32
ITERS = (CHUNKS + NW - 1) // NW


def _body(x_hbm, y_hbm, z_hbm, b_hbm, tab_hbm, out_hbm,
          xv, yv, zv, bv, idxv, gv, sem_in, sem_g):
    w = lax.axis_index("s") * 2 + lax.axis_index("c")

    def start_in(k, p):
        c = w + NW * k

        @pl.when(c < CHUNKS)
        def _():
            sl = pl.ds(c * C, C)
            pltpu.async_copy(x_hbm.at[sl], xv[p], sem_in[p])
            pltpu.async_copy(y_hbm.at[sl], yv[p], sem_in[p])
            pltpu.async_copy(z_hbm.at[sl], zv[p], sem_in[p])
            pltpu.async_copy(b_hbm.at[sl], bv[p], sem_in[p])

    def wait_in(k, p):
        c = w + NW * k

        @pl.when(c < CHUNKS)
        def _():
            sl = pl.ds(0, C)
            pltpu.make_async_copy(x_hbm.at[sl], xv[p], sem_in[p]).wait()
            pltpu.make_async_copy(y_hbm.at[sl], yv[p], sem_in[p]).wait()
            pltpu.make_async_copy(z_hbm.at[sl], zv[p], sem_in[p]).wait()
            pltpu.make_async_copy(b_hbm.at[sl], bv[p], sem_in[p]).wait()

    def compute(k, p):
        c = w + NW * k

        @pl.when(c < CHUNKS)
        def _():
            def step(i, carry):
                s = pl.ds(i * 16, 16)
                xf = xv[p][s]
                yf = yv[p][s]
                zf = zv[p][s]
                bi = bv[p][s]
                gx = ((xf * 0.5 + 0.5) * 128.0).astype(jnp.int32)
                gy = ((yf * 0.5 + 0.5) * 128.0).astype(jnp.int32)
                gz = ((zf * 0.5 + 0.5) * 128.0).astype(jnp.int32)
                gx = jnp.minimum(jnp.maximum(gx, 0), 127)
                gy = jnp.minimum(jnp.maximum(gy, 0), 127)
                gz = jnp.minimum(jnp.maximum(gz, 0), 127)
                f = ((bi * 128 + gx) * 128 + gy) * 128 + gz
                idxv[p][s] = f
                return carry

            lax.fori_loop(0, C // 16, step, 0)

    def fire_gather(k, p):
        c = w + NW * k

        @pl.when(c < CHUNKS)
        def _():
            pltpu.async_copy(tab_hbm.at[idxv[p]], gv[p], sem_g[p])

    def drain_gather(k, p):
        c = w + NW * k

        @pl.when(c < CHUNKS)
        def _():
            pltpu.make_async_copy(tab_hbm.at[idxv[p]], gv[p], sem_g[p]).wait()
            pltpu.sync_copy(gv[p], out_hbm.at[pl.ds(c * C, C)])

    start_in(0, 0)

    def outer(it, carry):
        for p in (0, 1):
            k = 2 * it + p
            wait_in(k, p)
            compute(k, p)

            @pl.when(k > 0)
            def _():
                drain_gather(k - 1, p ^ 1)

            fire_gather(k, p)
            start_in(k + 1, p ^ 1)
        return carry

    lax.fori_loop(0, ITERS // 2, outer, 0)
    drain_gather(ITERS - 1, (ITERS - 1) & 1)


def kernel(pts, bidx, occ_grid_per_batch, resolution):
    x = pts[:, 0]
    y = pts[:, 1]
    z = pts[:, 2]
    tab = occ_grid_per_batch.reshape(-1)

    mesh = plsc.VectorSubcoreMesh(core_axis_name="c", subcore_axis_name="s")
    out = pl.kernel(
        _body,
        out_type=jax.ShapeDtypeStruct((N,), jnp.bool_),
        mesh=mesh,
        scratch_types=[
            [pltpu.VMEM((C,), jnp.float32)] * 2,
            [pltpu.VMEM((C,), jnp.float32)] * 2,
            [pltpu.VMEM((C,), jnp.float32)] * 2,
            [pltpu.VMEM((C,), jnp.int32)] * 2,
            [pltpu.VMEM((C,), jnp.int32)] * 2,
            [pltpu.VMEM((C,), jnp.bool_)] * 2,
            [pltpu.SemaphoreType.DMA] * 2,
            [pltpu.SemaphoreType.DMA] * 2,
        ],
    )(x, y, z, bidx, tab)
    return out
